# baseline (speedup 1.0000x reference)
import functools

import jax
import jax.numpy as jnp
from jax import lax
from jax.experimental import pallas as pl
from jax.experimental.pallas import tpu as pltpu
from jax.experimental.pallas import tpu_sc as plsc

NUM_CORES = 2
NUM_SUBCORES = 16
NUM_WORKERS = NUM_CORES * NUM_SUBCORES

CHUNK = 128


def _sc_gather(tab128, idx_t, *, n_tok, seq, dim, row_w):
    toks_per_w = n_tok // NUM_WORKERS
    j_per_s = toks_per_w // CHUNK
    n_chunks = seq * j_per_s
    n_body = n_chunks // 2 - 1

    mesh = plsc.VectorSubcoreMesh(core_axis_name="c", subcore_axis_name="s")

    @functools.partial(
        pl.kernel,
        mesh=mesh,
        out_type=jax.ShapeDtypeStruct((n_tok, seq, row_w), jnp.float32),
        compiler_params=pltpu.CompilerParams(use_tc_tiling_on_sc=True,
                                             needs_layout_passes=False),
        scratch_types=[
            pltpu.VMEM((seq, toks_per_w), jnp.int32),
            pltpu.VMEM((CHUNK, 1, row_w), jnp.float32),
            pltpu.VMEM((CHUNK, 1, row_w), jnp.float32),
            pltpu.VMEM((CHUNK, 1, row_w), jnp.float32),
            pltpu.VMEM((CHUNK, 1, row_w), jnp.float32),
            pltpu.SemaphoreType.DMA,
            pltpu.SemaphoreType.DMA,
            pltpu.SemaphoreType.DMA,
            pltpu.SemaphoreType.DMA,
            pltpu.SemaphoreType.DMA,
            pltpu.SemaphoreType.DMA,
            pltpu.SemaphoreType.DMA,
            pltpu.SemaphoreType.DMA,
        ],
    )
    def k(tab_hbm, idx_hbm, out_hbm, idx_v, buf_a, buf_b, buf_c, buf_d,
          gsem_a, gsem_b, gsem_c, gsem_d, ssem_a, ssem_b, ssem_c, ssem_d):
        wid = lax.axis_index("s") * NUM_CORES + lax.axis_index("c")
        tok_base = wid * toks_per_w

        pltpu.sync_copy(idx_hbm.at[:, pl.ds(tok_base, toks_per_w)], idx_v)

        def fire_gather(buf, gsem, c):
            s = c // j_per_s
            j = lax.rem(c, j_per_s)
            pltpu.async_copy(
                tab_hbm.at[idx_v.at[s, pl.ds(j * CHUNK, CHUNK)]],
                buf.at[:, 0, :], gsem)

        def wait_gather(buf, gsem):
            pltpu.make_async_copy(tab_hbm.at[pl.ds(0, CHUNK)],
                                  buf.at[:, 0, :], gsem).wait()

        def store_pair(buf, c):
            s = c // j_per_s
            j = lax.rem(c, j_per_s)
            src = buf
            dst = out_hbm.at[pl.ds(tok_base + j * CHUNK, CHUNK),
                             pl.ds(s, 1), :]
            return src, dst

        def fire_store(buf, ssem, c):
            src, dst = store_pair(buf, c)
            pltpu.async_copy(src, dst, ssem)

        def wait_store(buf, ssem, c):
            src, dst = store_pair(buf, c)
            pltpu.make_async_copy(src, dst, ssem).wait()

        ring = [(buf_a, gsem_a, ssem_a), (buf_b, gsem_b, ssem_b),
                (buf_c, gsem_c, ssem_c), (buf_d, gsem_d, ssem_d)]
        nbuf = len(ring)
        for b, (buf, gsem, _) in enumerate(ring):
            fire_gather(buf, gsem, b)

        def body(t, _):
            c0 = nbuf * t
            for b, (buf, gsem, ssem) in enumerate(ring):
                c = c0 + b
                wait_gather(buf, gsem)
                fire_store(buf, ssem, c)
                wait_store(buf, ssem, c)
                fire_gather(buf, gsem, c + nbuf)
            return _

        lax.fori_loop(0, n_chunks // nbuf - 1, body, 0)

        c0 = n_chunks - nbuf
        for b, (buf, gsem, ssem) in enumerate(ring):
            c = c0 + b
            wait_gather(buf, gsem)
            fire_store(buf, ssem, c)
            wait_store(buf, ssem, c)

    return k(tab128, idx_t)


def kernel(token_ids, embeddings):
    n_tok, seq = token_ids.shape
    n_emb, dim = embeddings.shape
    row_w = 2 * dim
    tab128 = jnp.pad(embeddings, ((0, 0), (0, row_w - dim)))
    idx_t = token_ids.astype(jnp.int32).T
    out = _sc_gather(tab128, idx_t, n_tok=n_tok, seq=seq, dim=dim,
                     row_w=row_w)
    return out[:, :, :dim]
